# baseline (speedup 1.0000x reference)
import jax
import jax.numpy as jnp
from jax import lax
from jax.experimental import pallas as pl
from jax.experimental.pallas import tpu as pltpu
from jax.experimental.pallas import tpu_sc as plsc

B = 4096
K = 200
D = 32
NN = 10
NU = 1000000
NI = 1000000
W1 = 1e-07
W2 = 1.0
W3 = 1e-07
W4 = 1.0
NEG_WEIGHT = 200.0
GAMMA = 1e-04
LM = 2.75

NC = 2
NS = 16
L = 16
NW = NC * NS
BPW = B // NW
CH = 128
NBUF = 4

NEG_CHUNKS = BPW * K // CH
NBH_CHUNKS = BPW * NN // CH

NORM_BLK = 256000
NORM_GRID = NU * D // NORM_BLK


def _iota16():
    return lax.iota(jnp.int32, L)


def _sc_body(users, pos, negf, GuTf, Gi, bu, bi, nmTf, cmTf,
             pos_s_o, pos_w_o, neg_s_o, neg_w_o, sc_s_o, sim_o,
             uidx_v, pidx_v, uidx32_v, urowsT_v, prows_v, buv, bipv,
             negblk_v, c0, c1, c2, c3, g0, g1, g2, g3, b0, b1, b2, b3,
             tmpS, tmpW, sfull_v, wfull_v,
             semA, semB, semC, semD, semN, semU, semP, semBU, semBI,
             semS1, semS2):
    wid = lax.axis_index("s") * NC + lax.axis_index("c")
    base = pl.multiple_of(wid * BPW, BPW)
    cbufs = (c0, c1, c2, c3)
    gbufs = (g0, g1, g2, g3)
    bbufs = (b0, b1, b2, b3)
    sems = (semA, semB, semC, semD)

    pltpu.sync_copy(users.at[pl.ds(base, BPW)], uidx_v)
    pltpu.sync_copy(pos.at[pl.ds(base, BPW)], pidx_v)
    d0 = pltpu.async_copy(
        negf.at[pl.ds(pl.multiple_of(base * K, 8), BPW * K)], negblk_v, semN)

    def uidx_row(d, _):
        def grp(g, _):
            uidx32_v[d, pl.ds(g * L, L)] = \
                uidx_v[pl.ds(g * L, L)] + d * NU
            return 0
        lax.fori_loop(0, BPW // L, grp, 0)
        return 0
    lax.fori_loop(0, D, uidx_row, 0)
    dus = [pltpu.async_copy(GuTf.at[uidx32_v.at[d]],
                            urowsT_v.at[d, pl.ds(0, BPW)], semU)
           for d in range(D)]

    dp = pltpu.async_copy(Gi.at[pidx_v], prows_v, semP)
    dbu = pltpu.async_copy(bu.at[uidx_v], buv.at[pl.ds(0, BPW)], semBU)
    dbi = pltpu.async_copy(bi.at[pidx_v], bipv, semBI)
    for du in dus:
        du.wait()
    dp.wait()
    dbu.wait()
    dbi.wait()

    def pos_group(g, _):
        jv = g * L + _iota16()
        acc0 = jnp.zeros((L,), jnp.float32)
        acc1 = jnp.zeros((L,), jnp.float32)
        for d in range(D):
            dv = jnp.full((L,), d, jnp.int32)
            term = (plsc.load_gather(prows_v, [jv, dv])
                    * plsc.load_gather(urowsT_v, [dv, jv]))
            if d % 2 == 0:
                acc0 = acc0 + term
            else:
                acc1 = acc1 + term
        tmpS[pl.ds(g * L, L)] = acc0 + acc1
        w = W1 + W2 * buv[pl.ds(g * L, L)] * bipv[pl.ds(g * L, L)]
        tmpW[pl.ds(g * L, L)] = w
        return 0
    lax.fori_loop(0, BPW // L, pos_group, 0)
    pltpu.sync_copy(tmpS, pos_s_o.at[pl.ds(base, BPW)])
    pltpu.sync_copy(tmpW, pos_w_o.at[pl.ds(base, BPW)])

    def nbh_chunk(t, _):
        jbase = t * CH

        def build_idx(g, _):
            jv = jbase + g * L + _iota16()
            ev = lax.div(jv, NN)
            rv = lax.rem(jv, NN)
            pid = plsc.load_gather(pidx_v, [ev])
            c0[pl.ds(g * L, L)] = pid + rv * NI
            return 0
        lax.fori_loop(0, CH // L, build_idx, 0)

        dn = pltpu.async_copy(nmTf.at[c0], c1, semS1)
        dsim = pltpu.async_copy(cmTf.at[c0], tmpW, semS2)
        dn.wait()
        pltpu.async_copy(Gi.at[c1], g0, semS1).wait()

        def dot_group(g, _):
            jloc = g * L + _iota16()
            ev = lax.div(jbase + jloc, NN)
            acc0 = jnp.zeros((L,), jnp.float32)
            acc1 = jnp.zeros((L,), jnp.float32)
            for d in range(D):
                dv = jnp.full((L,), d, jnp.int32)
                term = (plsc.load_gather(g0, [jloc, dv])
                        * plsc.load_gather(urowsT_v, [dv, ev]))
                if d % 2 == 0:
                    acc0 = acc0 + term
                else:
                    acc1 = acc1 + term
            tmpS[pl.ds(g * L, L)] = acc0 + acc1
            return 0
        lax.fori_loop(0, CH // L, dot_group, 0)

        dsim.wait()
        off = pl.multiple_of(base * NN + jbase, 8)
        pltpu.sync_copy(tmpS, sc_s_o.at[pl.ds(off, CH)])
        pltpu.sync_copy(tmpW, sim_o.at[pl.ds(off, CH)])
        return 0
    lax.fori_loop(0, NBH_CHUNKS, nbh_chunk, 0)

    d0.wait()

    def fire_neg(t, cbuf, gbuf, bbuf, sem):
        def bld(g, _):
            jv = t * CH + g * L + _iota16()
            cbuf[pl.ds(g * L, L)] = plsc.load_gather(negblk_v, [jv])
            return 0
        lax.fori_loop(0, CH // L, bld, 0)
        return (pltpu.async_copy(Gi.at[cbuf], gbuf, sem),
                pltpu.async_copy(bi.at[cbuf], bbuf, sem))

    def compute_neg(t, gbuf, bbuf):
        e0 = lax.div(t * CH, K)
        cut = (e0 + 1) * K - t * CH
        uw = [urowsT_v[d, pl.ds(e0, L)] for d in range(D)]
        u0s = [w[0] for w in uw]
        u1s = [w[1] for w in uw]
        buw = buv[pl.ds(e0, L)]
        bu0 = buw[0]
        bu1 = buw[1]

        def group(g, _):
            jv = g * L + _iota16()
            m = jv < cut
            acc0 = jnp.zeros((L,), jnp.float32)
            acc1 = jnp.zeros((L,), jnp.float32)
            acc2 = jnp.zeros((L,), jnp.float32)
            acc3 = jnp.zeros((L,), jnp.float32)
            for d in range(D):
                dv = jnp.full((L,), d, jnp.int32)
                ub = jnp.where(m, u0s[d], u1s[d])
                term = plsc.load_gather(gbuf, [jv, dv]) * ub
                if d % 4 == 0:
                    acc0 = acc0 + term
                elif d % 4 == 1:
                    acc1 = acc1 + term
                elif d % 4 == 2:
                    acc2 = acc2 + term
                else:
                    acc3 = acc3 + term
            sfull_v[pl.ds(t * CH + g * L, L)] = (acc0 + acc1) + (acc2 + acc3)
            busel = jnp.where(m, bu0, bu1)
            wfull_v[pl.ds(t * CH + g * L, L)] = \
                W3 + W4 * busel * bbuf[pl.ds(g * L, L)]
            return 0
        lax.fori_loop(0, CH // L, group, 0)

    def neg_quad(q, _):
        t0 = NBUF * q
        ds = [fire_neg(t0 + p, cbufs[p], gbufs[p], bbufs[p], sems[p])
              for p in range(NBUF)]
        for p in range(NBUF):
            ds[p][0].wait()
            ds[p][1].wait()
            compute_neg(t0 + p, gbufs[p], bbufs[p])
        return 0
    lax.fori_loop(0, NEG_CHUNKS // NBUF, neg_quad, 0)

    off = pl.multiple_of(base * K, 8)
    pltpu.sync_copy(sfull_v, neg_s_o.at[pl.ds(off, BPW * K)])
    pltpu.sync_copy(wfull_v, neg_w_o.at[pl.ds(off, BPW * K)])


@jax.jit
def _sc_call(users, pos, negf, GuTf, Gi, bu, bi, nmTf, cmTf):
    mesh = plsc.VectorSubcoreMesh(core_axis_name="c", subcore_axis_name="s")
    f32 = jnp.float32
    i32 = jnp.int32
    out_type = (
        jax.ShapeDtypeStruct((B,), f32),
        jax.ShapeDtypeStruct((B,), f32),
        jax.ShapeDtypeStruct((B * K,), f32),
        jax.ShapeDtypeStruct((B * K,), f32),
        jax.ShapeDtypeStruct((B * NN,), f32),
        jax.ShapeDtypeStruct((B * NN,), f32),
    )
    scratch = [
        pltpu.VMEM((BPW,), i32),
        pltpu.VMEM((BPW,), i32),
        pltpu.VMEM((D, BPW), i32),
        pltpu.VMEM((D, BPW + L), f32),
        pltpu.VMEM((BPW, D), f32),
        pltpu.VMEM((BPW + L,), f32),
        pltpu.VMEM((BPW,), f32),
        pltpu.VMEM((BPW * K,), i32),
        pltpu.VMEM((CH,), i32),
        pltpu.VMEM((CH,), i32),
        pltpu.VMEM((CH,), i32),
        pltpu.VMEM((CH,), i32),
        pltpu.VMEM((CH, D), f32),
        pltpu.VMEM((CH, D), f32),
        pltpu.VMEM((CH, D), f32),
        pltpu.VMEM((CH, D), f32),
        pltpu.VMEM((CH,), f32),
        pltpu.VMEM((CH,), f32),
        pltpu.VMEM((CH,), f32),
        pltpu.VMEM((CH,), f32),
        pltpu.VMEM((CH,), f32),
        pltpu.VMEM((CH,), f32),
        pltpu.VMEM((BPW * K,), f32),
        pltpu.VMEM((BPW * K,), f32),
        pltpu.SemaphoreType.DMA,
        pltpu.SemaphoreType.DMA,
        pltpu.SemaphoreType.DMA,
        pltpu.SemaphoreType.DMA,
        pltpu.SemaphoreType.DMA,
        pltpu.SemaphoreType.DMA,
        pltpu.SemaphoreType.DMA,
        pltpu.SemaphoreType.DMA,
        pltpu.SemaphoreType.DMA,
        pltpu.SemaphoreType.DMA,
        pltpu.SemaphoreType.DMA,
    ]
    return pl.kernel(
        _sc_body, out_type=out_type, mesh=mesh, scratch_types=scratch,
        compiler_params=pltpu.CompilerParams(
            needs_layout_passes=False, use_tc_tiling_on_sc=False),
    )(users, pos, negf, GuTf, Gi, bu, bi, nmTf, cmTf)


def _softplus(x):
    return jnp.maximum(x, 0.0) + jnp.log1p(jnp.exp(-jnp.abs(x)))


def _norm_body(guf, gif, out, accs):
    i = pl.program_id(0)

    @pl.when(i == 0)
    def _init():
        accs[0] = 0.0

    gu = guf[...]
    gi = gif[...]
    accs[0] += jnp.sum(gu * gu) + jnp.sum(gi * gi)

    @pl.when(i == NORM_GRID - 1)
    def _fini():
        out[...] = jnp.reshape(accs[0], (1, 1))


@jax.jit
def _norm_call(GuTf, GiTf):
    return pl.pallas_call(
        _norm_body,
        grid=(NORM_GRID,),
        in_specs=[
            pl.BlockSpec((NORM_BLK,), lambda i: (i,)),
            pl.BlockSpec((NORM_BLK,), lambda i: (i,)),
        ],
        out_specs=pl.BlockSpec((1, 1), lambda i: (0, 0)),
        out_shape=jax.ShapeDtypeStruct((1, 1), jnp.float32),
        scratch_shapes=[pltpu.SMEM((1,), jnp.float32)],
    )(GuTf, GiTf)


def _tc_body(ps, pw, ns, nw, ss, sim, out):
    out[...] = jnp.reshape(
        jnp.sum(pw[...] * _softplus(-ps[...]))
        + LM * jnp.sum(sim[...] * _softplus(-ss[...]))
        + (NEG_WEIGHT / K) * jnp.sum(nw[...] * _softplus(ns[...])),
        (1, 1))


@jax.jit
def _tc_call(ps, pw, ns, nw, ss, sim):
    return pl.pallas_call(
        _tc_body,
        grid=(1,),
        in_specs=[
            pl.BlockSpec((32, 128), lambda i: (0, 0)),
            pl.BlockSpec((32, 128), lambda i: (0, 0)),
            pl.BlockSpec((6400, 128), lambda i: (0, 0)),
            pl.BlockSpec((6400, 128), lambda i: (0, 0)),
            pl.BlockSpec((320, 128), lambda i: (0, 0)),
            pl.BlockSpec((320, 128), lambda i: (0, 0)),
        ],
        out_specs=pl.BlockSpec((1, 1), lambda i: (0, 0)),
        out_shape=jax.ShapeDtypeStruct((1, 1), jnp.float32),
    )(ps, pw, ns, nw, ss, sim)


def kernel(users, pos_items, neg_items, Gu, Gi, beta_uD, beta_iD,
           ii_neighbor_mat, ii_constraint_mat):
    users = users.astype(jnp.int32)
    pos = pos_items.astype(jnp.int32)
    negf = neg_items.reshape(-1).astype(jnp.int32)
    nmat = ii_neighbor_mat.astype(jnp.int32)

    GuTf = jnp.swapaxes(Gu, 0, 1).reshape(-1)
    GiTf = jnp.swapaxes(Gi, 0, 1).reshape(-1)
    nmTf = jnp.swapaxes(nmat, 0, 1).reshape(-1)
    cmTf = jnp.swapaxes(ii_constraint_mat, 0, 1).reshape(-1)

    ps, pw, nsc, nwt, ssc, sim = _sc_call(
        users, pos, negf, GuTf, Gi, beta_uD, beta_iD, nmTf, cmTf)

    sumsq = _norm_call(GuTf, GiTf)

    out = _tc_call(
        ps.reshape(32, 128),
        pw.reshape(32, 128),
        nsc.reshape(6400, 128),
        nwt.reshape(6400, 128),
        ssc.reshape(320, 128),
        sim.reshape(320, 128),
    )
    return out[0, 0] + (GAMMA * 0.5) * sumsq[0, 0]


# speedup vs baseline: 2.6647x; 2.6647x over previous
import jax
import jax.numpy as jnp
from jax import lax
from jax.experimental import pallas as pl
from jax.experimental.pallas import tpu as pltpu
from jax.experimental.pallas import tpu_sc as plsc

B = 4096
K = 200
D = 32
NN = 10
NU = 1000000
NI = 1000000
W1 = 1e-07
W2 = 1.0
W3 = 1e-07
W4 = 1.0
NEG_WEIGHT = 200.0
GAMMA = 1e-04
LM = 2.75

NC = 2
NS = 16
L = 16
NW = NC * NS
BPW = B // NW
CH = 128
NBUF = 4

NEG_CHUNKS = BPW * K // CH
NBH_CHUNKS = BPW * NN // CH

NORM_BLK = 256000
NORM_GRID = NU * D // NORM_BLK


def _iota16():
    return lax.iota(jnp.int32, L)


def _sc_body(users, pos, negf, Gu, Gi, bu, bi, nmatf, cmatf,
             pos_s_o, pos_w_o, neg_s_o, neg_w_o, sc_s_o, sim_o,
             uidx_v, pidx_v, urows_v, prows_v, buv, bipv,
             negblk_v, c0, c1, c2, c3, g0, g1, g2, g3, b0, b1, b2, b3,
             tmpS, tmpW, sfull_v, wfull_v,
             semA, semB, semC, semD, semN, semU, semP, semBU, semBI,
             semS1, semS2):
    wid = lax.axis_index("s") * NC + lax.axis_index("c")
    base = pl.multiple_of(wid * BPW, BPW)
    cbufs = (c0, c1, c2, c3)
    gbufs = (g0, g1, g2, g3)
    bbufs = (b0, b1, b2, b3)
    sems = (semA, semB, semC, semD)

    pltpu.sync_copy(users.at[pl.ds(base, BPW)], uidx_v)
    pltpu.sync_copy(pos.at[pl.ds(base, BPW)], pidx_v)
    d0 = pltpu.async_copy(
        negf.at[pl.ds(pl.multiple_of(base * K, 8), BPW * K)], negblk_v, semN)

    du = pltpu.async_copy(Gu.at[uidx_v], urows_v, semU)
    dp = pltpu.async_copy(Gi.at[pidx_v], prows_v, semP)
    dbu = pltpu.async_copy(bu.at[uidx_v], buv.at[pl.ds(0, BPW)], semBU)
    dbi = pltpu.async_copy(bi.at[pidx_v], bipv, semBI)
    du.wait()
    dp.wait()
    dbu.wait()
    dbi.wait()

    def pos_group(g, _):
        jv = g * L + _iota16()
        acc0 = jnp.zeros((L,), jnp.float32)
        acc1 = jnp.zeros((L,), jnp.float32)
        for d in range(D):
            dv = jnp.full((L,), d, jnp.int32)
            term = (plsc.load_gather(prows_v, [jv, dv])
                    * plsc.load_gather(urows_v, [jv, dv]))
            if d % 2 == 0:
                acc0 = acc0 + term
            else:
                acc1 = acc1 + term
        tmpS[pl.ds(g * L, L)] = acc0 + acc1
        w = W1 + W2 * buv[pl.ds(g * L, L)] * bipv[pl.ds(g * L, L)]
        tmpW[pl.ds(g * L, L)] = w
        return 0
    lax.fori_loop(0, BPW // L, pos_group, 0)
    pltpu.sync_copy(tmpS, pos_s_o.at[pl.ds(base, BPW)])
    pltpu.sync_copy(tmpW, pos_w_o.at[pl.ds(base, BPW)])

    def nbh_chunk(t, _):
        jbase = t * CH

        def build_idx(g, _):
            jv = jbase + g * L + _iota16()
            ev = lax.div(jv, NN)
            rv = lax.rem(jv, NN)
            pid = plsc.load_gather(pidx_v, [ev])
            c0[pl.ds(g * L, L)] = pid * NN + rv
            return 0
        lax.fori_loop(0, CH // L, build_idx, 0)

        dn = pltpu.async_copy(nmatf.at[c0], c1, semS1)
        dsim = pltpu.async_copy(cmatf.at[c0], tmpW, semS2)
        dn.wait()
        pltpu.async_copy(Gi.at[c1], g0, semS1).wait()

        def dot_group(g, _):
            jloc = g * L + _iota16()
            ev = lax.div(jbase + jloc, NN)
            acc0 = jnp.zeros((L,), jnp.float32)
            acc1 = jnp.zeros((L,), jnp.float32)
            for d in range(D):
                dv = jnp.full((L,), d, jnp.int32)
                term = (plsc.load_gather(g0, [jloc, dv])
                        * plsc.load_gather(urows_v, [ev, dv]))
                if d % 2 == 0:
                    acc0 = acc0 + term
                else:
                    acc1 = acc1 + term
            tmpS[pl.ds(g * L, L)] = acc0 + acc1
            return 0
        lax.fori_loop(0, CH // L, dot_group, 0)

        dsim.wait()
        off = pl.multiple_of(base * NN + jbase, 8)
        pltpu.sync_copy(tmpS, sc_s_o.at[pl.ds(off, CH)])
        pltpu.sync_copy(tmpW, sim_o.at[pl.ds(off, CH)])
        return 0
    lax.fori_loop(0, NBH_CHUNKS, nbh_chunk, 0)

    d0.wait()

    def fire_neg(t, cbuf, gbuf, bbuf, sem):
        def bld(g, _):
            jv = t * CH + g * L + _iota16()
            cbuf[pl.ds(g * L, L)] = plsc.load_gather(negblk_v, [jv])
            return 0
        lax.fori_loop(0, CH // L, bld, 0)
        return (pltpu.async_copy(Gi.at[cbuf], gbuf, sem),
                pltpu.async_copy(bi.at[cbuf], bbuf, sem))

    def compute_neg(t, gbuf, bbuf):
        e0 = lax.div(t * CH, K)
        e1 = jnp.minimum(e0 + 1, BPW - 1)
        cut = (e0 + 1) * K - t * CH
        u0h = (urows_v[e0, pl.ds(0, L)], urows_v[e0, pl.ds(L, L)])
        u1h = (urows_v[e1, pl.ds(0, L)], urows_v[e1, pl.ds(L, L)])
        u0s = [u0h[d // L][d % L] for d in range(D)]
        u1s = [u1h[d // L][d % L] for d in range(D)]
        buw = buv[pl.ds(e0, L)]
        bu0 = buw[0]
        bu1 = buw[1]

        def group(g, _):
            jv = g * L + _iota16()
            m = jv < cut
            acc0 = jnp.zeros((L,), jnp.float32)
            acc1 = jnp.zeros((L,), jnp.float32)
            acc2 = jnp.zeros((L,), jnp.float32)
            acc3 = jnp.zeros((L,), jnp.float32)
            for d in range(D):
                dv = jnp.full((L,), d, jnp.int32)
                ub = jnp.where(m, u0s[d], u1s[d])
                term = plsc.load_gather(gbuf, [jv, dv]) * ub
                if d % 4 == 0:
                    acc0 = acc0 + term
                elif d % 4 == 1:
                    acc1 = acc1 + term
                elif d % 4 == 2:
                    acc2 = acc2 + term
                else:
                    acc3 = acc3 + term
            sfull_v[pl.ds(t * CH + g * L, L)] = (acc0 + acc1) + (acc2 + acc3)
            busel = jnp.where(m, bu0, bu1)
            wfull_v[pl.ds(t * CH + g * L, L)] = \
                W3 + W4 * busel * bbuf[pl.ds(g * L, L)]
            return 0
        lax.fori_loop(0, CH // L, group, 0)

    def neg_quad(q, _):
        t0 = NBUF * q
        ds = [fire_neg(t0 + p, cbufs[p], gbufs[p], bbufs[p], sems[p])
              for p in range(NBUF)]
        for p in range(NBUF):
            ds[p][0].wait()
            ds[p][1].wait()
            compute_neg(t0 + p, gbufs[p], bbufs[p])
        return 0
    lax.fori_loop(0, NEG_CHUNKS // NBUF, neg_quad, 0)

    off = pl.multiple_of(base * K, 8)
    pltpu.sync_copy(sfull_v, neg_s_o.at[pl.ds(off, BPW * K)])
    pltpu.sync_copy(wfull_v, neg_w_o.at[pl.ds(off, BPW * K)])


@jax.jit
def _sc_call(users, pos, negf, Gu, Gi, bu, bi, nmatf, cmatf):
    mesh = plsc.VectorSubcoreMesh(core_axis_name="c", subcore_axis_name="s")
    f32 = jnp.float32
    i32 = jnp.int32
    out_type = (
        jax.ShapeDtypeStruct((B,), f32),
        jax.ShapeDtypeStruct((B,), f32),
        jax.ShapeDtypeStruct((B * K,), f32),
        jax.ShapeDtypeStruct((B * K,), f32),
        jax.ShapeDtypeStruct((B * NN,), f32),
        jax.ShapeDtypeStruct((B * NN,), f32),
    )
    scratch = [
        pltpu.VMEM((BPW,), i32),
        pltpu.VMEM((BPW,), i32),
        pltpu.VMEM((BPW, D), f32),
        pltpu.VMEM((BPW, D), f32),
        pltpu.VMEM((BPW + L,), f32),
        pltpu.VMEM((BPW,), f32),
        pltpu.VMEM((BPW * K,), i32),
        pltpu.VMEM((CH,), i32),
        pltpu.VMEM((CH,), i32),
        pltpu.VMEM((CH,), i32),
        pltpu.VMEM((CH,), i32),
        pltpu.VMEM((CH, D), f32),
        pltpu.VMEM((CH, D), f32),
        pltpu.VMEM((CH, D), f32),
        pltpu.VMEM((CH, D), f32),
        pltpu.VMEM((CH,), f32),
        pltpu.VMEM((CH,), f32),
        pltpu.VMEM((CH,), f32),
        pltpu.VMEM((CH,), f32),
        pltpu.VMEM((CH,), f32),
        pltpu.VMEM((CH,), f32),
        pltpu.VMEM((BPW * K,), f32),
        pltpu.VMEM((BPW * K,), f32),
        pltpu.SemaphoreType.DMA,
        pltpu.SemaphoreType.DMA,
        pltpu.SemaphoreType.DMA,
        pltpu.SemaphoreType.DMA,
        pltpu.SemaphoreType.DMA,
        pltpu.SemaphoreType.DMA,
        pltpu.SemaphoreType.DMA,
        pltpu.SemaphoreType.DMA,
        pltpu.SemaphoreType.DMA,
        pltpu.SemaphoreType.DMA,
        pltpu.SemaphoreType.DMA,
    ]
    return pl.kernel(
        _sc_body, out_type=out_type, mesh=mesh, scratch_types=scratch,
        compiler_params=pltpu.CompilerParams(
            needs_layout_passes=False, use_tc_tiling_on_sc=False),
    )(users, pos, negf, Gu, Gi, bu, bi, nmatf, cmatf)


def _softplus(x):
    return jnp.maximum(x, 0.0) + jnp.log1p(jnp.exp(-jnp.abs(x)))


NORM_RBLK = 8000
NORM_GRID2 = NU // NORM_RBLK


def _norm_body(gu, gi, out, accs):
    i = pl.program_id(0)

    @pl.when(i == 0)
    def _init():
        accs[0] = 0.0

    g = gu[...]
    h = gi[...]
    accs[0] += jnp.sum(g * g) + jnp.sum(h * h)

    @pl.when(i == NORM_GRID2 - 1)
    def _fini():
        out[...] = jnp.reshape(accs[0], (1, 1))


@jax.jit
def _norm_call(Gu, Gi):
    return pl.pallas_call(
        _norm_body,
        grid=(NORM_GRID2,),
        in_specs=[
            pl.BlockSpec((NORM_RBLK, D), lambda i: (i, 0)),
            pl.BlockSpec((NORM_RBLK, D), lambda i: (i, 0)),
        ],
        out_specs=pl.BlockSpec((1, 1), lambda i: (0, 0)),
        out_shape=jax.ShapeDtypeStruct((1, 1), jnp.float32),
        scratch_shapes=[pltpu.SMEM((1,), jnp.float32)],
    )(Gu, Gi)


def _tc_body(ps, pw, ns, nw, ss, sim, out):
    out[...] = jnp.reshape(
        jnp.sum(pw[...] * _softplus(-ps[...]))
        + LM * jnp.sum(sim[...] * _softplus(-ss[...]))
        + (NEG_WEIGHT / K) * jnp.sum(nw[...] * _softplus(ns[...])),
        (1, 1))


@jax.jit
def _tc_call(ps, pw, ns, nw, ss, sim):
    return pl.pallas_call(
        _tc_body,
        grid=(1,),
        in_specs=[
            pl.BlockSpec((32, 128), lambda i: (0, 0)),
            pl.BlockSpec((32, 128), lambda i: (0, 0)),
            pl.BlockSpec((6400, 128), lambda i: (0, 0)),
            pl.BlockSpec((6400, 128), lambda i: (0, 0)),
            pl.BlockSpec((320, 128), lambda i: (0, 0)),
            pl.BlockSpec((320, 128), lambda i: (0, 0)),
        ],
        out_specs=pl.BlockSpec((1, 1), lambda i: (0, 0)),
        out_shape=jax.ShapeDtypeStruct((1, 1), jnp.float32),
    )(ps, pw, ns, nw, ss, sim)


def kernel(users, pos_items, neg_items, Gu, Gi, beta_uD, beta_iD,
           ii_neighbor_mat, ii_constraint_mat):
    users = users.astype(jnp.int32)
    pos = pos_items.astype(jnp.int32)
    negf = neg_items.reshape(-1).astype(jnp.int32)
    nmat = ii_neighbor_mat.astype(jnp.int32)

    nmatf = nmat.reshape(-1)
    cmatf = ii_constraint_mat.reshape(-1)

    ps, pw, nsc, nwt, ssc, sim = _sc_call(
        users, pos, negf, Gu, Gi, beta_uD, beta_iD, nmatf, cmatf)

    sumsq = _norm_call(Gu, Gi)

    out = _tc_call(
        ps.reshape(32, 128),
        pw.reshape(32, 128),
        nsc.reshape(6400, 128),
        nwt.reshape(6400, 128),
        ssc.reshape(320, 128),
        sim.reshape(320, 128),
    )
    return out[0, 0] + (GAMMA * 0.5) * sumsq[0, 0]


# speedup vs baseline: 2.8887x; 1.0841x over previous
import jax
import jax.numpy as jnp
from jax import lax
from jax.experimental import pallas as pl
from jax.experimental.pallas import tpu as pltpu
from jax.experimental.pallas import tpu_sc as plsc

B = 4096
K = 200
D = 32
NN = 10
NU = 1000000
NI = 1000000
W1 = 1e-07
W2 = 1.0
W3 = 1e-07
W4 = 1.0
NEG_WEIGHT = 200.0
GAMMA = 1e-04
LM = 2.75

NC = 2
NS = 16
L = 16
NW = NC * NS
BPW = B // NW
CH = 128
NBUF = 8

NEG_CHUNKS = BPW * K // CH
NBH_CHUNKS = BPW * NN // CH

NORM_BLK = 256000
NORM_GRID = NU * D // NORM_BLK


def _iota16():
    return lax.iota(jnp.int32, L)


def _sc_body(users, pos, negf, Gu, Gi, bu, bi, nmatf, cmatf,
             pos_s_o, pos_w_o, neg_s_o, neg_w_o, sc_s_o, sim_o,
             uidx_v, pidx_v, urows_v, prows_v, buv, bipv,
             negblk_v, c0, c1, c2, c3, c4, c5, c6, c7,
             g0, g1, g2, g3, g4, g5, g6, g7,
             b0, b1, b2, b3, b4, b5, b6, b7,
             tmpS, tmpW, sfull_v, wfull_v,
             semA, semB, semC, semD, semE, semF, semG, semH,
             semN, semU, semP, semBU, semBI, semS1, semS2):
    wid = lax.axis_index("s") * NC + lax.axis_index("c")
    base = pl.multiple_of(wid * BPW, BPW)
    cbufs = (c0, c1, c2, c3, c4, c5, c6, c7)
    gbufs = (g0, g1, g2, g3, g4, g5, g6, g7)
    bbufs = (b0, b1, b2, b3, b4, b5, b6, b7)
    sems = (semA, semB, semC, semD, semE, semF, semG, semH)

    pltpu.sync_copy(users.at[pl.ds(base, BPW)], uidx_v)
    pltpu.sync_copy(pos.at[pl.ds(base, BPW)], pidx_v)
    d0 = pltpu.async_copy(
        negf.at[pl.ds(pl.multiple_of(base * K, 8), BPW * K)], negblk_v, semN)

    du = pltpu.async_copy(Gu.at[uidx_v], urows_v, semU)
    dp = pltpu.async_copy(Gi.at[pidx_v], prows_v, semP)
    dbu = pltpu.async_copy(bu.at[uidx_v], buv.at[pl.ds(0, BPW)], semBU)
    dbi = pltpu.async_copy(bi.at[pidx_v], bipv, semBI)
    du.wait()
    dp.wait()
    dbu.wait()
    dbi.wait()

    def pos_group(g, _):
        jv = g * L + _iota16()
        acc0 = jnp.zeros((L,), jnp.float32)
        acc1 = jnp.zeros((L,), jnp.float32)
        for d in range(D):
            dv = jnp.full((L,), d, jnp.int32)
            term = (plsc.load_gather(prows_v, [jv, dv])
                    * plsc.load_gather(urows_v, [jv, dv]))
            if d % 2 == 0:
                acc0 = acc0 + term
            else:
                acc1 = acc1 + term
        tmpS[pl.ds(g * L, L)] = acc0 + acc1
        w = W1 + W2 * buv[pl.ds(g * L, L)] * bipv[pl.ds(g * L, L)]
        tmpW[pl.ds(g * L, L)] = w
        return 0
    lax.fori_loop(0, BPW // L, pos_group, 0)
    pltpu.sync_copy(tmpS, pos_s_o.at[pl.ds(base, BPW)])
    pltpu.sync_copy(tmpW, pos_w_o.at[pl.ds(base, BPW)])

    def nbh_chunk(t, _):
        jbase = t * CH

        def build_idx(g, _):
            jv = jbase + g * L + _iota16()
            ev = lax.div(jv, NN)
            rv = lax.rem(jv, NN)
            pid = plsc.load_gather(pidx_v, [ev])
            c0[pl.ds(g * L, L)] = pid * NN + rv
            return 0
        lax.fori_loop(0, CH // L, build_idx, 0)

        dn = pltpu.async_copy(nmatf.at[c0], c1, semS1)
        dsim = pltpu.async_copy(cmatf.at[c0], tmpW, semS2)
        dn.wait()
        pltpu.async_copy(Gi.at[c1], g0, semS1).wait()

        def dot_group(g, _):
            jloc = g * L + _iota16()
            ev = lax.div(jbase + jloc, NN)
            acc0 = jnp.zeros((L,), jnp.float32)
            acc1 = jnp.zeros((L,), jnp.float32)
            for d in range(D):
                dv = jnp.full((L,), d, jnp.int32)
                term = (plsc.load_gather(g0, [jloc, dv])
                        * plsc.load_gather(urows_v, [ev, dv]))
                if d % 2 == 0:
                    acc0 = acc0 + term
                else:
                    acc1 = acc1 + term
            tmpS[pl.ds(g * L, L)] = acc0 + acc1
            return 0
        lax.fori_loop(0, CH // L, dot_group, 0)

        dsim.wait()
        off = pl.multiple_of(base * NN + jbase, 8)
        pltpu.sync_copy(tmpS, sc_s_o.at[pl.ds(off, CH)])
        pltpu.sync_copy(tmpW, sim_o.at[pl.ds(off, CH)])
        return 0
    lax.fori_loop(0, NBH_CHUNKS, nbh_chunk, 0)

    d0.wait()

    def fire_neg(t, cbuf, gbuf, bbuf, sem):
        def bld(g, _):
            jv = t * CH + g * L + _iota16()
            cbuf[pl.ds(g * L, L)] = plsc.load_gather(negblk_v, [jv])
            return 0
        lax.fori_loop(0, CH // L, bld, 0)
        return (pltpu.async_copy(Gi.at[cbuf], gbuf, sem),
                pltpu.async_copy(bi.at[cbuf], bbuf, sem))

    def compute_neg(t, gbuf, bbuf):
        e0 = lax.div(t * CH, K)
        e1 = jnp.minimum(e0 + 1, BPW - 1)
        cut = (e0 + 1) * K - t * CH
        u0h = (urows_v[e0, pl.ds(0, L)], urows_v[e0, pl.ds(L, L)])
        u1h = (urows_v[e1, pl.ds(0, L)], urows_v[e1, pl.ds(L, L)])
        u0s = [u0h[d // L][d % L] for d in range(D)]
        u1s = [u1h[d // L][d % L] for d in range(D)]
        buw = buv[pl.ds(e0, L)]
        bu0 = buw[0]
        bu1 = buw[1]

        def group(g, _):
            jv = g * L + _iota16()
            m = jv < cut
            acc0 = jnp.zeros((L,), jnp.float32)
            acc1 = jnp.zeros((L,), jnp.float32)
            acc2 = jnp.zeros((L,), jnp.float32)
            acc3 = jnp.zeros((L,), jnp.float32)
            for d in range(D):
                dv = jnp.full((L,), d, jnp.int32)
                ub = jnp.where(m, u0s[d], u1s[d])
                term = plsc.load_gather(gbuf, [jv, dv]) * ub
                if d % 4 == 0:
                    acc0 = acc0 + term
                elif d % 4 == 1:
                    acc1 = acc1 + term
                elif d % 4 == 2:
                    acc2 = acc2 + term
                else:
                    acc3 = acc3 + term
            sfull_v[pl.ds(t * CH + g * L, L)] = (acc0 + acc1) + (acc2 + acc3)
            busel = jnp.where(m, bu0, bu1)
            wfull_v[pl.ds(t * CH + g * L, L)] = \
                W3 + W4 * busel * bbuf[pl.ds(g * L, L)]
            return 0
        lax.fori_loop(0, CH // L, group, 0)

    def neg_quad(q, _):
        t0 = NBUF * q
        ds = [fire_neg(t0 + p, cbufs[p], gbufs[p], bbufs[p], sems[p])
              for p in range(NBUF)]
        for p in range(NBUF):
            ds[p][0].wait()
            ds[p][1].wait()
            compute_neg(t0 + p, gbufs[p], bbufs[p])
        return 0
    lax.fori_loop(0, NEG_CHUNKS // NBUF, neg_quad, 0)

    off = pl.multiple_of(base * K, 8)
    pltpu.sync_copy(sfull_v, neg_s_o.at[pl.ds(off, BPW * K)])
    pltpu.sync_copy(wfull_v, neg_w_o.at[pl.ds(off, BPW * K)])


@jax.jit
def _sc_call(users, pos, negf, Gu, Gi, bu, bi, nmatf, cmatf):
    mesh = plsc.VectorSubcoreMesh(core_axis_name="c", subcore_axis_name="s")
    f32 = jnp.float32
    i32 = jnp.int32
    out_type = (
        jax.ShapeDtypeStruct((B,), f32),
        jax.ShapeDtypeStruct((B,), f32),
        jax.ShapeDtypeStruct((B * K,), f32),
        jax.ShapeDtypeStruct((B * K,), f32),
        jax.ShapeDtypeStruct((B * NN,), f32),
        jax.ShapeDtypeStruct((B * NN,), f32),
    )
    scratch = [
        pltpu.VMEM((BPW,), i32),
        pltpu.VMEM((BPW,), i32),
        pltpu.VMEM((BPW, D), f32),
        pltpu.VMEM((BPW, D), f32),
        pltpu.VMEM((BPW + L,), f32),
        pltpu.VMEM((BPW,), f32),
        pltpu.VMEM((BPW * K,), i32),
    ] + [pltpu.VMEM((CH,), i32) for _ in range(NBUF)] \
      + [pltpu.VMEM((CH, D), f32) for _ in range(NBUF)] \
      + [pltpu.VMEM((CH,), f32) for _ in range(NBUF)] + [
        pltpu.VMEM((CH,), f32),
        pltpu.VMEM((CH,), f32),
        pltpu.VMEM((BPW * K,), f32),
        pltpu.VMEM((BPW * K,), f32),
    ] + [pltpu.SemaphoreType.DMA for _ in range(NBUF)] + [
        pltpu.SemaphoreType.DMA,
        pltpu.SemaphoreType.DMA,
        pltpu.SemaphoreType.DMA,
        pltpu.SemaphoreType.DMA,
        pltpu.SemaphoreType.DMA,
        pltpu.SemaphoreType.DMA,
        pltpu.SemaphoreType.DMA,
    ]
    return pl.kernel(
        _sc_body, out_type=out_type, mesh=mesh, scratch_types=scratch,
        compiler_params=pltpu.CompilerParams(
            needs_layout_passes=False, use_tc_tiling_on_sc=False),
    )(users, pos, negf, Gu, Gi, bu, bi, nmatf, cmatf)


def _softplus(x):
    return jnp.maximum(x, 0.0) + jnp.log1p(jnp.exp(-jnp.abs(x)))


TBL_ROWS = 250000
TC_GRID = 50
TBL_BLK = TBL_ROWS // TC_GRID


def _tc_body(gu, gi, ps, pw, ns, nw, ss, sim, out, accs):
    i = pl.program_id(0)

    @pl.when(i == 0)
    def _init():
        accs[0] = jnp.sum(pw[...] * _softplus(-ps[...])) \
            + LM * jnp.sum(sim[...] * _softplus(-ss[...]))
        accs[1] = 0.0

    accs[0] += (NEG_WEIGHT / K) * jnp.sum(nw[...] * _softplus(ns[...]))
    accs[1] += jnp.sum(gu[...] * gu[...]) + jnp.sum(gi[...] * gi[...])

    @pl.when(i == TC_GRID - 1)
    def _fini():
        out[...] = jnp.reshape(accs[0] + (GAMMA * 0.5) * accs[1], (1, 1))


@jax.jit
def _tc_call(GuR, GiR, ps, pw, ns, nw, ss, sim):
    return pl.pallas_call(
        _tc_body,
        grid=(TC_GRID,),
        in_specs=[
            pl.BlockSpec((TBL_BLK, 128), lambda i: (i, 0)),
            pl.BlockSpec((TBL_BLK, 128), lambda i: (i, 0)),
            pl.BlockSpec((32, 128), lambda i: (0, 0)),
            pl.BlockSpec((32, 128), lambda i: (0, 0)),
            pl.BlockSpec((1, 128, 128), lambda i: (i, 0, 0)),
            pl.BlockSpec((1, 128, 128), lambda i: (i, 0, 0)),
            pl.BlockSpec((320, 128), lambda i: (0, 0)),
            pl.BlockSpec((320, 128), lambda i: (0, 0)),
        ],
        out_specs=pl.BlockSpec((1, 1), lambda i: (0, 0)),
        out_shape=jax.ShapeDtypeStruct((1, 1), jnp.float32),
        scratch_shapes=[pltpu.SMEM((2,), jnp.float32)],
    )(GuR, GiR, ps, pw, ns, nw, ss, sim)


def kernel(users, pos_items, neg_items, Gu, Gi, beta_uD, beta_iD,
           ii_neighbor_mat, ii_constraint_mat):
    users = users.astype(jnp.int32)
    pos = pos_items.astype(jnp.int32)
    negf = neg_items.reshape(-1).astype(jnp.int32)
    nmat = ii_neighbor_mat.astype(jnp.int32)

    nmatf = nmat.reshape(-1)
    cmatf = ii_constraint_mat.reshape(-1)

    ps, pw, nsc, nwt, ssc, sim = _sc_call(
        users, pos, negf, Gu, Gi, beta_uD, beta_iD, nmatf, cmatf)

    out = _tc_call(
        Gu.reshape(TBL_ROWS, 128),
        Gi.reshape(TBL_ROWS, 128),
        ps.reshape(32, 128),
        pw.reshape(32, 128),
        nsc.reshape(TC_GRID, 128, 128),
        nwt.reshape(TC_GRID, 128, 128),
        ssc.reshape(320, 128),
        sim.reshape(320, 128),
    )
    return out[0, 0]


# speedup vs baseline: 7.0469x; 2.4394x over previous
import jax
import jax.numpy as jnp
from jax import lax
from jax.experimental import pallas as pl
from jax.experimental.pallas import tpu as pltpu
from jax.experimental.pallas import tpu_sc as plsc

B = 4096
K = 200
D = 32
NN = 10
NU = 1000000
NI = 1000000
W1 = 1e-07
W2 = 1.0
W3 = 1e-07
W4 = 1.0
NEG_WEIGHT = 200.0
GAMMA = 1e-04
LM = 2.75

NC = 2
NS = 16
L = 16
NW = NC * NS
BPW = B // NW
CH = 128
NBUF = 8

NEG_CHUNKS = BPW * K // CH
NBH_CHUNKS = BPW * NN // CH

NORM_BLK = 256000
NORM_GRID = NU * D // NORM_BLK


def _iota16():
    return lax.iota(jnp.int32, L)


def _sc_body(users, pos, negf, urowsA, Gi, bu, bi, nidxf,
             pos_s_o, pos_w_o, neg_s_o, neg_w_o, sc_s_o,
             uidx_v, pidx_v, urows_v, prows_v, buv, bipv,
             negblk_v, c0, c1, c2, c3, c4, c5, c6, c7,
             g0, g1, g2, g3, g4, g5, g6, g7,
             b0, b1, b2, b3, b4, b5, b6, b7,
             tmpS, tmpW, sfull_v, wfull_v,
             semA, semB, semC, semD, semE, semF, semG, semH,
             semN, semU, semP, semBU, semBI, semS1, semS2):
    wid = lax.axis_index("s") * NC + lax.axis_index("c")
    base = pl.multiple_of(wid * BPW, BPW)
    cbufs = (c0, c1, c2, c3, c4, c5, c6, c7)
    gbufs = (g0, g1, g2, g3, g4, g5, g6, g7)
    bbufs = (b0, b1, b2, b3, b4, b5, b6, b7)
    sems = (semA, semB, semC, semD, semE, semF, semG, semH)

    pltpu.sync_copy(users.at[pl.ds(base, BPW)], uidx_v)
    pltpu.sync_copy(pos.at[pl.ds(base, BPW)], pidx_v)
    d0 = pltpu.async_copy(
        negf.at[pl.ds(pl.multiple_of(base * K, 8), BPW * K)], negblk_v, semN)

    du = pltpu.async_copy(urowsA.at[pl.ds(base, BPW)], urows_v, semU)
    dp = pltpu.async_copy(Gi.at[pidx_v], prows_v, semP)
    dbu = pltpu.async_copy(bu.at[uidx_v], buv.at[pl.ds(0, BPW)], semBU)
    dbi = pltpu.async_copy(bi.at[pidx_v], bipv, semBI)
    du.wait()
    dp.wait()
    dbu.wait()
    dbi.wait()

    def pos_group(g, _):
        jv = g * L + _iota16()
        acc0 = jnp.zeros((L,), jnp.float32)
        acc1 = jnp.zeros((L,), jnp.float32)
        for d in range(D):
            dv = jnp.full((L,), d, jnp.int32)
            term = (plsc.load_gather(prows_v, [jv, dv])
                    * plsc.load_gather(urows_v, [jv, dv]))
            if d % 2 == 0:
                acc0 = acc0 + term
            else:
                acc1 = acc1 + term
        tmpS[pl.ds(g * L, L)] = acc0 + acc1
        w = W1 + W2 * buv[pl.ds(g * L, L)] * bipv[pl.ds(g * L, L)]
        tmpW[pl.ds(g * L, L)] = w
        return 0
    lax.fori_loop(0, BPW // L, pos_group, 0)
    pltpu.sync_copy(tmpS, pos_s_o.at[pl.ds(base, BPW)])
    pltpu.sync_copy(tmpW, pos_w_o.at[pl.ds(base, BPW)])

    def nbh_chunk(t, _):
        jbase = t * CH
        off = pl.multiple_of(base * NN + jbase, 8)
        pltpu.sync_copy(nidxf.at[pl.ds(off, CH)], c1)
        pltpu.async_copy(Gi.at[c1], g0, semS1).wait()

        def dot_group(g, _):
            jloc = g * L + _iota16()
            ev = lax.div(jbase + jloc, NN)
            acc0 = jnp.zeros((L,), jnp.float32)
            acc1 = jnp.zeros((L,), jnp.float32)
            for d in range(D):
                dv = jnp.full((L,), d, jnp.int32)
                term = (plsc.load_gather(g0, [jloc, dv])
                        * plsc.load_gather(urows_v, [ev, dv]))
                if d % 2 == 0:
                    acc0 = acc0 + term
                else:
                    acc1 = acc1 + term
            tmpS[pl.ds(g * L, L)] = acc0 + acc1
            return 0
        lax.fori_loop(0, CH // L, dot_group, 0)

        pltpu.sync_copy(tmpS, sc_s_o.at[pl.ds(off, CH)])
        return 0
    lax.fori_loop(0, NBH_CHUNKS, nbh_chunk, 0)

    d0.wait()

    def fire_neg(t, cbuf, gbuf, bbuf, sem):
        def bld(g, _):
            jv = t * CH + g * L + _iota16()
            cbuf[pl.ds(g * L, L)] = plsc.load_gather(negblk_v, [jv])
            return 0
        lax.fori_loop(0, CH // L, bld, 0)
        return (pltpu.async_copy(Gi.at[cbuf], gbuf, sem),
                pltpu.async_copy(bi.at[cbuf], bbuf, sem))

    def compute_neg(t, gbuf, bbuf):
        e0 = lax.div(t * CH, K)
        e1 = jnp.minimum(e0 + 1, BPW - 1)
        cut = (e0 + 1) * K - t * CH
        u0h = (urows_v[e0, pl.ds(0, L)], urows_v[e0, pl.ds(L, L)])
        u1h = (urows_v[e1, pl.ds(0, L)], urows_v[e1, pl.ds(L, L)])
        u0s = [u0h[d // L][d % L] for d in range(D)]
        u1s = [u1h[d // L][d % L] for d in range(D)]
        buw = buv[pl.ds(e0, L)]
        bu0 = buw[0]
        bu1 = buw[1]

        def group(g, _):
            jv = g * L + _iota16()
            m = jv < cut
            acc0 = jnp.zeros((L,), jnp.float32)
            acc1 = jnp.zeros((L,), jnp.float32)
            acc2 = jnp.zeros((L,), jnp.float32)
            acc3 = jnp.zeros((L,), jnp.float32)
            for d in range(D):
                dv = jnp.full((L,), d, jnp.int32)
                ub = jnp.where(m, u0s[d], u1s[d])
                term = plsc.load_gather(gbuf, [jv, dv]) * ub
                if d % 4 == 0:
                    acc0 = acc0 + term
                elif d % 4 == 1:
                    acc1 = acc1 + term
                elif d % 4 == 2:
                    acc2 = acc2 + term
                else:
                    acc3 = acc3 + term
            sfull_v[pl.ds(t * CH + g * L, L)] = (acc0 + acc1) + (acc2 + acc3)
            busel = jnp.where(m, bu0, bu1)
            wfull_v[pl.ds(t * CH + g * L, L)] = \
                W3 + W4 * busel * bbuf[pl.ds(g * L, L)]
            return 0
        lax.fori_loop(0, CH // L, group, 0)

    def neg_quad(q, _):
        t0 = NBUF * q
        ds = [fire_neg(t0 + p, cbufs[p], gbufs[p], bbufs[p], sems[p])
              for p in range(NBUF)]
        for p in range(NBUF):
            ds[p][0].wait()
            ds[p][1].wait()
            compute_neg(t0 + p, gbufs[p], bbufs[p])
        return 0
    lax.fori_loop(0, NEG_CHUNKS // NBUF, neg_quad, 0)

    off = pl.multiple_of(base * K, 8)
    pltpu.sync_copy(sfull_v, neg_s_o.at[pl.ds(off, BPW * K)])
    pltpu.sync_copy(wfull_v, neg_w_o.at[pl.ds(off, BPW * K)])


@jax.jit
def _sc_call(users, pos, negf, urowsA, Gi, bu, bi, nidxf):
    mesh = plsc.VectorSubcoreMesh(core_axis_name="c", subcore_axis_name="s")
    f32 = jnp.float32
    i32 = jnp.int32
    out_type = (
        jax.ShapeDtypeStruct((B,), f32),
        jax.ShapeDtypeStruct((B,), f32),
        jax.ShapeDtypeStruct((B * K,), f32),
        jax.ShapeDtypeStruct((B * K,), f32),
        jax.ShapeDtypeStruct((B * NN,), f32),
    )
    scratch = [
        pltpu.VMEM((BPW,), i32),
        pltpu.VMEM((BPW,), i32),
        pltpu.VMEM((BPW, D), f32),
        pltpu.VMEM((BPW, D), f32),
        pltpu.VMEM((BPW + L,), f32),
        pltpu.VMEM((BPW,), f32),
        pltpu.VMEM((BPW * K,), i32),
    ] + [pltpu.VMEM((CH,), i32) for _ in range(NBUF)] \
      + [pltpu.VMEM((CH, D), f32) for _ in range(NBUF)] \
      + [pltpu.VMEM((CH,), f32) for _ in range(NBUF)] + [
        pltpu.VMEM((CH,), f32),
        pltpu.VMEM((CH,), f32),
        pltpu.VMEM((BPW * K,), f32),
        pltpu.VMEM((BPW * K,), f32),
    ] + [pltpu.SemaphoreType.DMA for _ in range(NBUF)] + [
        pltpu.SemaphoreType.DMA,
        pltpu.SemaphoreType.DMA,
        pltpu.SemaphoreType.DMA,
        pltpu.SemaphoreType.DMA,
        pltpu.SemaphoreType.DMA,
        pltpu.SemaphoreType.DMA,
        pltpu.SemaphoreType.DMA,
    ]
    return pl.kernel(
        _sc_body, out_type=out_type, mesh=mesh, scratch_types=scratch,
        compiler_params=pltpu.CompilerParams(
            needs_layout_passes=False, use_tc_tiling_on_sc=False),
    )(users, pos, negf, urowsA, Gi, bu, bi, nidxf)


def _softplus(x):
    return jnp.maximum(x, 0.0) + jnp.log1p(jnp.exp(-jnp.abs(x)))


TC_GRID = 50


def _normT_body(gt, out, accs):
    i = pl.program_id(0)

    @pl.when(i == 0)
    def _init():
        accs[0] = 0.0

    g = gt[...]
    accs[0] += jnp.sum(g * g)

    @pl.when(i == D // 8 - 1)
    def _fini():
        out[...] = jnp.reshape(accs[0], (1, 1))


@jax.jit
def _normT_call(GT):
    return pl.pallas_call(
        _normT_body,
        grid=(D // 8,),
        in_specs=[pl.BlockSpec((8, NU), lambda i: (i, 0))],
        out_specs=pl.BlockSpec((1, 1), lambda i: (0, 0)),
        out_shape=jax.ShapeDtypeStruct((1, 1), jnp.float32),
        scratch_shapes=[pltpu.SMEM((1,), jnp.float32)],
        compiler_params=pltpu.CompilerParams(
            vmem_limit_bytes=120 * 1024 * 1024),
    )(GT)


def _tc_body(ps, pw, ns, nw, ss, sim, out):
    out[...] = jnp.reshape(
        jnp.sum(pw[...] * _softplus(-ps[...]))
        + LM * jnp.sum(sim[...] * _softplus(-ss[...]))
        + (NEG_WEIGHT / K) * jnp.sum(nw[...] * _softplus(ns[...])),
        (1, 1))


@jax.jit
def _tc_call(ps, pw, ns, nw, ss, sim):
    return pl.pallas_call(
        _tc_body,
        grid=(1,),
        in_specs=[
            pl.BlockSpec((32, 128), lambda i: (0, 0)),
            pl.BlockSpec((32, 128), lambda i: (0, 0)),
            pl.BlockSpec((6400, 128), lambda i: (0, 0)),
            pl.BlockSpec((6400, 128), lambda i: (0, 0)),
            pl.BlockSpec((320, 128), lambda i: (0, 0)),
            pl.BlockSpec((320, 128), lambda i: (0, 0)),
        ],
        out_specs=pl.BlockSpec((1, 1), lambda i: (0, 0)),
        out_shape=jax.ShapeDtypeStruct((1, 1), jnp.float32),
    )(ps, pw, ns, nw, ss, sim)


def kernel(users, pos_items, neg_items, Gu, Gi, beta_uD, beta_iD,
           ii_neighbor_mat, ii_constraint_mat):
    users = users.astype(jnp.int32)
    pos = pos_items.astype(jnp.int32)
    negf = neg_items.reshape(-1).astype(jnp.int32)

    urowsA = Gu[users]
    nidxf = ii_neighbor_mat[pos].reshape(-1).astype(jnp.int32)
    sim = ii_constraint_mat[pos].reshape(-1)

    ps, pw, nsc, nwt, ssc = _sc_call(
        users, pos, negf, urowsA, Gi, beta_uD, beta_iD, nidxf)

    sumsq = _normT_call(jnp.swapaxes(Gu, 0, 1))[0, 0] \
        + _normT_call(jnp.swapaxes(Gi, 0, 1))[0, 0]

    out = _tc_call(
        ps.reshape(32, 128),
        pw.reshape(32, 128),
        nsc.reshape(6400, 128),
        nwt.reshape(6400, 128),
        ssc.reshape(320, 128),
        sim.reshape(320, 128),
    )
    return out[0, 0] + (GAMMA * 0.5) * sumsq
